# baseline (speedup 1.0000x reference)
import functools

import jax
import jax.numpy as jnp
from jax import lax
from jax.experimental import pallas as pl
from jax.experimental.pallas import tpu as pltpu
from jax.experimental.pallas import tpu_sc as plsc

B = 32
D_MODEL = 4096
D_FF = 4096
NUM_EXPERTS = 8
TF = 256
NF = D_FF // TF

_SC_INFO = plsc.get_sparse_core_info()
_NC = _SC_INFO.num_cores
_LANES = _SC_INFO.num_lanes
_HALF = _LANES * NUM_EXPERTS


def _gate_kernel(x_ref, gw_ref, out_ref):
    out_ref[:] = jnp.dot(x_ref[:], gw_ref[:],
                         preferred_element_type=jnp.float32)


def _route_sc_kernel(logits_hbm, out_hbm, lv, dv):
    wid = lax.axis_index("s") * _NC + lax.axis_index("c")

    @pl.when(wid < B // _LANES)
    def _():
        pltpu.sync_copy(logits_hbm.at[pl.ds(wid * _HALF, _HALF)], lv)
        iota = jax.lax.broadcasted_iota(jnp.int32, (_LANES,), 0)
        idx = [iota * NUM_EXPERTS + e for e in range(NUM_EXPERTS)]
        v = [plsc.load_gather(lv, [idx[e]]) for e in range(NUM_EXPERTS)]

        m1 = v[0]
        i1 = jnp.zeros((_LANES,), jnp.int32)
        for e in range(1, NUM_EXPERTS):
            upd = v[e] > m1
            m1 = jnp.where(upd, v[e], m1)
            i1 = jnp.where(upd, e, i1)
        m2 = jnp.full((_LANES,), -jnp.inf, jnp.float32)
        i2 = jnp.zeros((_LANES,), jnp.int32)
        for e in range(NUM_EXPERTS):
            upd = (i1 != e) & (v[e] > m2)
            m2 = jnp.where(upd, v[e], m2)
            i2 = jnp.where(upd, e, i2)

        wa = 1.0 / (1.0 + jnp.exp(m2 - m1))
        wb = 1.0 - wa
        zero = jnp.zeros((_LANES,), jnp.float32)
        for e in range(NUM_EXPERTS):
            val = jnp.where(i1 == e, wa, jnp.where(i2 == e, wb, zero))
            plsc.store_scatter(dv, [idx[e]], val)
        pltpu.sync_copy(dv, out_hbm.at[pl.ds(wid * _HALF, _HALF)])


_route_sc = functools.partial(
    pl.kernel,
    out_type=jax.ShapeDtypeStruct((B * NUM_EXPERTS,), jnp.float32),
    mesh=plsc.VectorSubcoreMesh(core_axis_name="c", subcore_axis_name="s"),
    scratch_types=[
        pltpu.VMEM((_HALF,), jnp.float32),
        pltpu.VMEM((_HALF,), jnp.float32),
    ],
    compiler_params=pltpu.CompilerParams(needs_layout_passes=False),
)(_route_sc_kernel)


def _moe_kernel(x_ref, w1_ref, w3_ref, w2_ref, out_ref):
    f = pl.program_id(1)

    x = x_ref[:]
    h1 = jnp.dot(x, w1_ref[0], preferred_element_type=jnp.float32)
    h3 = jnp.dot(x, w3_ref[0], preferred_element_type=jnp.float32)
    h = (h1 * jax.nn.sigmoid(h1)) * h3
    p = jnp.dot(h, w2_ref[0], preferred_element_type=jnp.float32)

    @pl.when(f == 0)
    def _first():
        out_ref[0] = p

    @pl.when(f > 0)
    def _rest():
        out_ref[0] += p


def _combine_kernel(eo_ref, dw_ref, out_ref):
    dw = dw_ref[:]
    acc = eo_ref[0] * dw[:, 0:1]
    for e in range(1, NUM_EXPERTS):
        acc += eo_ref[e] * dw[:, e:e + 1]
    out_ref[:] = acc


@jax.jit
def kernel(x, gate_w, w1, w3, w2):
    tokens = x.reshape(B, D_MODEL)

    logits = pl.pallas_call(
        _gate_kernel,
        out_shape=jax.ShapeDtypeStruct((B, NUM_EXPERTS), jnp.float32),
    )(tokens, gate_w)

    dense_w = _route_sc(logits.reshape(-1)).reshape(B, NUM_EXPERTS)

    expert_out = pl.pallas_call(
        _moe_kernel,
        grid=(NUM_EXPERTS, NF),
        in_specs=[
            pl.BlockSpec((B, D_MODEL), lambda e, f: (0, 0)),
            pl.BlockSpec((1, D_MODEL, TF), lambda e, f: (e, 0, f)),
            pl.BlockSpec((1, D_MODEL, TF), lambda e, f: (e, 0, f)),
            pl.BlockSpec((1, TF, D_MODEL), lambda e, f: (e, f, 0)),
        ],
        out_specs=pl.BlockSpec((1, B, D_MODEL), lambda e, f: (e, 0, 0)),
        out_shape=jax.ShapeDtypeStruct((NUM_EXPERTS, B, D_MODEL),
                                       jnp.float32),
    )(tokens, w1, w3, w2)

    out = pl.pallas_call(
        _combine_kernel,
        out_shape=jax.ShapeDtypeStruct((B, D_MODEL), jnp.float32),
    )(expert_out, dense_w)
    return out.reshape(B, 1, 1, D_MODEL)


# speedup vs baseline: 1.0125x; 1.0125x over previous
import functools

import jax
import jax.numpy as jnp
from jax import lax
from jax.experimental import pallas as pl
from jax.experimental.pallas import tpu as pltpu
from jax.experimental.pallas import tpu_sc as plsc

B = 32
D_MODEL = 4096
D_FF = 4096
NUM_EXPERTS = 8
TF = 256
NF = D_FF // TF

_SC_INFO = plsc.get_sparse_core_info()
_NC = _SC_INFO.num_cores
_LANES = _SC_INFO.num_lanes
_HALF = _LANES * NUM_EXPERTS


def _gate_kernel(x_ref, gw_ref, out_ref):
    out_ref[:] = jnp.dot(x_ref[:], gw_ref[:],
                         preferred_element_type=jnp.float32)


def _route_sc_kernel(logits_hbm, out_hbm, lv, dv):
    wid = lax.axis_index("s") * _NC + lax.axis_index("c")

    @pl.when(wid < B // _LANES)
    def _():
        pltpu.sync_copy(logits_hbm.at[pl.ds(wid * _HALF, _HALF)], lv)
        iota = jax.lax.broadcasted_iota(jnp.int32, (_LANES,), 0)
        idx = [iota * NUM_EXPERTS + e for e in range(NUM_EXPERTS)]
        v = [plsc.load_gather(lv, [idx[e]]) for e in range(NUM_EXPERTS)]

        m1 = v[0]
        i1 = jnp.zeros((_LANES,), jnp.int32)
        for e in range(1, NUM_EXPERTS):
            upd = v[e] > m1
            m1 = jnp.where(upd, v[e], m1)
            i1 = jnp.where(upd, e, i1)
        m2 = jnp.full((_LANES,), -jnp.inf, jnp.float32)
        i2 = jnp.zeros((_LANES,), jnp.int32)
        for e in range(NUM_EXPERTS):
            upd = (i1 != e) & (v[e] > m2)
            m2 = jnp.where(upd, v[e], m2)
            i2 = jnp.where(upd, e, i2)

        wa = 1.0 / (1.0 + jnp.exp(m2 - m1))
        wb = 1.0 - wa
        zero = jnp.zeros((_LANES,), jnp.float32)
        for e in range(NUM_EXPERTS):
            val = jnp.where(i1 == e, wa, jnp.where(i2 == e, wb, zero))
            plsc.store_scatter(dv, [idx[e]], val)
        pltpu.sync_copy(dv, out_hbm.at[pl.ds(wid * _HALF, _HALF)])


_route_sc = functools.partial(
    pl.kernel,
    out_type=jax.ShapeDtypeStruct((B * NUM_EXPERTS,), jnp.float32),
    mesh=plsc.VectorSubcoreMesh(core_axis_name="c", subcore_axis_name="s"),
    scratch_types=[
        pltpu.VMEM((_HALF,), jnp.float32),
        pltpu.VMEM((_HALF,), jnp.float32),
    ],
    compiler_params=pltpu.CompilerParams(needs_layout_passes=False),
)(_route_sc_kernel)


def _moe_kernel(x_ref, dw_ref, w1_ref, w3_ref, w2_ref, out_ref, acc_ref):
    e = pl.program_id(0)
    f = pl.program_id(1)

    @pl.when((e == 0) & (f == 0))
    def _init():
        acc_ref[:] = jnp.zeros_like(acc_ref)

    x = x_ref[:]
    h1 = jnp.dot(x, w1_ref[0], preferred_element_type=jnp.float32)
    h3 = jnp.dot(x, w3_ref[0], preferred_element_type=jnp.float32)
    h = (h1 * jax.nn.sigmoid(h1)) * h3
    ecols = jax.lax.broadcasted_iota(jnp.int32, (B, NUM_EXPERTS), 1)
    scale = jnp.sum(jnp.where(ecols == e, dw_ref[:], 0.0),
                    axis=1, keepdims=True)
    acc_ref[:] += jnp.dot(h * scale, w2_ref[0],
                          preferred_element_type=jnp.float32)

    @pl.when((e == NUM_EXPERTS - 1) & (f == NF - 1))
    def _done():
        out_ref[:] = acc_ref[:]


@jax.jit
def kernel(x, gate_w, w1, w3, w2):
    tokens = x.reshape(B, D_MODEL)

    logits = pl.pallas_call(
        _gate_kernel,
        out_shape=jax.ShapeDtypeStruct((B, NUM_EXPERTS), jnp.float32),
    )(tokens, gate_w)

    dense_w = _route_sc(logits.reshape(-1)).reshape(B, NUM_EXPERTS)

    out = pl.pallas_call(
        _moe_kernel,
        grid=(NUM_EXPERTS, NF),
        in_specs=[
            pl.BlockSpec((B, D_MODEL), lambda e, f: (0, 0)),
            pl.BlockSpec((B, NUM_EXPERTS), lambda e, f: (0, 0)),
            pl.BlockSpec((1, D_MODEL, TF), lambda e, f: (e, 0, f)),
            pl.BlockSpec((1, D_MODEL, TF), lambda e, f: (e, 0, f)),
            pl.BlockSpec((1, TF, D_MODEL), lambda e, f: (e, f, 0)),
        ],
        out_specs=pl.BlockSpec((B, D_MODEL), lambda e, f: (0, 0)),
        out_shape=jax.ShapeDtypeStruct((B, D_MODEL), jnp.float32),
        scratch_shapes=[
            pltpu.VMEM((B, D_MODEL), jnp.float32),
        ],
    )(tokens, dense_w, w1, w3, w2)
    return out.reshape(B, 1, 1, D_MODEL)
